# baseline (speedup 1.0000x reference)
import functools

import jax
import jax.numpy as jnp
from jax import lax
from jax.experimental import pallas as pl
from jax.experimental.pallas import tpu as pltpu
from jax.experimental.pallas import tpu_sc as plsc

NC = 2
NS = 16
L = 16
CE = 80
BI = 3
NW = NC * NS


def _mm_body(x_ref, w_ref, o_ref):
    o_ref[...] = jnp.dot(x_ref[...], w_ref[...],
                         preferred_element_type=jnp.float32)


def _mm(x, W, bm):
    n, d = x.shape
    return pl.pallas_call(
        _mm_body,
        grid=(n // bm,),
        in_specs=[pl.BlockSpec((bm, d), lambda i: (i, 0)),
                  pl.BlockSpec((d, d), lambda i: (0, 0))],
        out_specs=pl.BlockSpec((bm, d), lambda i: (i, 0)),
        out_shape=jax.ShapeDtypeStruct((n, d), jnp.float32),
    )(x, W)


def _mm_final_body(p_ref, x_ref, w_ref, o_ref):
    s = p_ref[0] + p_ref[1] - x_ref[...]
    o_ref[...] = jnp.dot(s, w_ref[...], preferred_element_type=jnp.float32)


def _mm_final(parts, x, W2, bm):
    n, d = x.shape
    return pl.pallas_call(
        _mm_final_body,
        grid=(n // bm,),
        in_specs=[pl.BlockSpec((NC, bm, d), lambda i: (0, i, 0)),
                  pl.BlockSpec((bm, d), lambda i: (i, 0)),
                  pl.BlockSpec((d, d), lambda i: (0, 0))],
        out_specs=pl.BlockSpec((bm, d), lambda i: (i, 0)),
        out_shape=jax.ShapeDtypeStruct((n, d), jnp.float32),
    )(parts, x, W2)


def _spmm_sc(x, eb, wb):
    n, d = x.shape
    ept = (eb.shape[0] - 1) * CE // NW
    cpt = ept // CE
    nb = (cpt - 1) // BI
    P = 80
    np_ = n // P
    mesh = plsc.VectorSubcoreMesh(core_axis_name="c", subcore_axis_name="s",
                                  num_cores=NC, num_subcores=NS)

    @functools.partial(
        pl.kernel,
        out_type=jax.ShapeDtypeStruct((NC, n, d), jnp.float32),
        mesh=mesh,
        scratch_types=[
            pltpu.VMEM((3, BI, 2, CE), jnp.int32),
            pltpu.VMEM((3, BI, 1, CE), jnp.float32),
            pltpu.VMEM((3, CE, d), jnp.float32),
            pltpu.VMEM_SHARED((n, d), jnp.float32),
            pltpu.SemaphoreType.DMA,
            pltpu.SemaphoreType.DMA,
            pltpu.SemaphoreType.DMA,
            pltpu.SemaphoreType.DMA,
            pltpu.SemaphoreType.DMA,
            pltpu.SemaphoreType.DMA,
            pltpu.SemaphoreType.DMA,
        ],
    )
    def spmm(x_hbm, eb_hbm, wb_hbm, out_hbm, ring, wring, rows, acc,
             sb, sg0, sg1, sg2, ss0, ss1, ss2):
        sg = (sg0, sg1, sg2)
        ss = (ss0, ss1, ss2)
        cid = lax.axis_index("c")
        sid = lax.axis_index("s")
        wid = sid * NC + cid

        zero = jnp.zeros((L,), jnp.float32)

        def zrow(i, carry):
            for g in range(d // L):
                rows[0, i, pl.ds(g * L, L)] = zero
            return carry

        lax.fori_loop(0, P, zrow, 0)
        p0 = sid * np_ // NS
        p1 = (sid + 1) * np_ // NS

        def zpiece(p, carry):
            pltpu.sync_copy(rows.at[0, pl.ds(0, P)], acc.at[pl.ds(p * P, P)])
            return carry

        lax.fori_loop(p0, p1, zpiece, 0)
        plsc.subcore_barrier()

        def fire_blk(g):
            base = wid * cpt + g * BI
            slot = lax.rem(g, 3)
            pltpu.async_copy(eb_hbm.at[pl.ds(base, BI)], ring.at[slot], sb)
            pltpu.async_copy(wb_hbm.at[pl.ds(base, BI)], wring.at[slot], sb)

        def drain_blk(g):
            slot = lax.rem(g, 3)
            pltpu.make_async_copy(eb_hbm.at[pl.ds(0, BI)],
                                  ring.at[slot], sb).wait()
            pltpu.make_async_copy(wb_hbm.at[pl.ds(0, BI)],
                                  wring.at[slot], sb).wait()

        def fire_gather(r, j, b):
            pltpu.async_copy(x_hbm.at[ring.at[r, j, 0]], rows.at[b], sg[b])

        def drain_gather(b):
            pltpu.make_async_copy(x_hbm.at[pl.ds(0, CE)],
                                  rows.at[b], sg[b]).wait()

        def fire_scatter(r, j, b):
            pltpu.async_copy(rows.at[b], acc.at[ring.at[r, j, 1]], ss[b],
                             add=True)

        def drain_scatter(b):
            pltpu.make_async_copy(x_hbm.at[pl.ds(0, CE)],
                                  rows.at[b], ss[b]).wait()

        def scale(r, j, b):
            def sgroup(t, carry):
                wg = wring[r, j, 0, pl.ds(t * L, L)]
                for jj in range(L):
                    w0 = wg[jj]
                    wv = jnp.full((L,), w0 + w0, jnp.float32)
                    for q in range(d // L):
                        sl = pl.ds(q * L, L)
                        rows[b, t * L + jj, sl] = rows[b, t * L + jj, sl] * wv
                return carry

            lax.fori_loop(0, CE // L, sgroup, 0)

        fire_blk(0)
        drain_blk(0)
        fire_gather(0, 0, 0)
        fire_blk(1)

        def body3(i, carry):
            r = lax.rem(i, 3)
            rn = lax.rem(i + 1, 3)

            @pl.when(i > 0)
            def _():
                drain_scatter(1)

            fire_gather(r, 1, 1)
            drain_gather(0)
            scale(r, 0, 0)
            fire_scatter(r, 0, 0)

            @pl.when(i > 0)
            def _():
                drain_scatter(2)

            fire_gather(r, 2, 2)
            drain_gather(1)
            scale(r, 1, 1)
            fire_scatter(r, 1, 1)

            drain_blk(i + 1)
            drain_scatter(0)

            @pl.when(i + 2 <= nb)
            def _():
                fire_blk(i + 2)

            fire_gather(rn, 0, 0)
            drain_gather(2)
            scale(r, 2, 2)
            fire_scatter(r, 2, 2)
            return carry

        lax.fori_loop(0, nb, body3, 0)
        rl = lax.rem(nb, 3)
        drain_scatter(1)
        fire_gather(rl, 1, 1)
        drain_gather(0)
        scale(rl, 0, 0)
        fire_scatter(rl, 0, 0)
        drain_scatter(2)
        drain_gather(1)
        scale(rl, 1, 1)
        fire_scatter(rl, 1, 1)
        drain_scatter(0)
        drain_scatter(1)
        plsc.subcore_barrier()

        def wpiece(p, carry):
            sl = pl.ds(p * P, P)
            pltpu.sync_copy(acc.at[sl], out_hbm.at[cid, sl])
            return carry

        lax.fori_loop(p0, p1, wpiece, 0)

    return spmm(x, eb, wb)


def kernel(input, edge_index, edge_weight, W):
    srcv = edge_index[1].reshape(-1, CE)
    dstv = edge_index[0].reshape(-1, CE)
    eb = jnp.stack([srcv, dstv], axis=1)
    eb = jnp.concatenate(
        [eb, jnp.zeros((1, 2, CE), jnp.int32)], axis=0)
    wb = edge_weight.reshape(-1, 1, CE)
    wb = jnp.concatenate(
        [wb, jnp.zeros((1, 1, CE), jnp.float32)], axis=0)
    parts = _spmm_sc(input, eb, wb)
    u = _mm_final(parts, input, W, 2000)
    return _mm(u, W, 2000)


# speedup vs baseline: 1.0584x; 1.0584x over previous
import functools

import jax
import jax.numpy as jnp
from jax import lax
from jax.experimental import pallas as pl
from jax.experimental.pallas import tpu as pltpu
from jax.experimental.pallas import tpu_sc as plsc

NC = 2
NS = 16
L = 16
CE = 80
BI = 3
NW = NC * NS


def _mm_body(x_ref, w_ref, o_ref):
    o_ref[...] = jnp.dot(x_ref[...], w_ref[...],
                         preferred_element_type=jnp.float32)


def _mm(x, W, bm):
    n, d = x.shape
    return pl.pallas_call(
        _mm_body,
        grid=(n // bm,),
        in_specs=[pl.BlockSpec((bm, d), lambda i: (i, 0)),
                  pl.BlockSpec((d, d), lambda i: (0, 0))],
        out_specs=pl.BlockSpec((bm, d), lambda i: (i, 0)),
        out_shape=jax.ShapeDtypeStruct((n, d), jnp.float32),
    )(x, W)


def _mm_final_body(p_ref, x_ref, w_ref, o_ref):
    s = p_ref[0] + p_ref[1] - x_ref[...]
    o_ref[...] = jnp.dot(s, w_ref[...], preferred_element_type=jnp.float32)


def _mm_final(parts, x, W2, bm):
    n, d = x.shape
    return pl.pallas_call(
        _mm_final_body,
        grid=(n // bm,),
        in_specs=[pl.BlockSpec((NC, bm, d), lambda i: (0, i, 0)),
                  pl.BlockSpec((bm, d), lambda i: (i, 0)),
                  pl.BlockSpec((d, d), lambda i: (0, 0))],
        out_specs=pl.BlockSpec((bm, d), lambda i: (i, 0)),
        out_shape=jax.ShapeDtypeStruct((n, d), jnp.float32),
    )(parts, x, W2)


def _spmm_sc(x, eb, wb):
    n, d = x.shape
    ept = (eb.shape[0] - 1) * CE // NW
    cpt = ept // CE
    nb = (cpt - 1) // BI
    P = 80
    np_ = n // P
    mesh = plsc.VectorSubcoreMesh(core_axis_name="c", subcore_axis_name="s",
                                  num_cores=NC, num_subcores=NS)

    @functools.partial(
        pl.kernel,
        out_type=jax.ShapeDtypeStruct((NC, n, d), jnp.float32),
        mesh=mesh,
        scratch_types=[
            pltpu.VMEM((3, BI, 2, CE), jnp.int32),
            pltpu.VMEM((3, BI, 1, CE), jnp.float32),
            pltpu.VMEM((3, CE, d), jnp.float32),
            pltpu.VMEM_SHARED((n, d), jnp.float32),
            pltpu.SemaphoreType.DMA,
            pltpu.SemaphoreType.DMA,
            pltpu.SemaphoreType.DMA,
            pltpu.SemaphoreType.DMA,
            pltpu.SemaphoreType.DMA,
            pltpu.SemaphoreType.DMA,
            pltpu.SemaphoreType.DMA,
        ],
    )
    def spmm(x_hbm, eb_hbm, wb_hbm, out_hbm, ring, wring, rows, acc,
             sb, sg0, sg1, sg2, ss0, ss1, ss2):
        sg = (sg0, sg1, sg2)
        ss = (ss0, ss1, ss2)
        cid = lax.axis_index("c")
        sid = lax.axis_index("s")
        wid = sid * NC + cid

        zero = jnp.zeros((L,), jnp.float32)

        def zrow(i, carry):
            for g in range(d // L):
                rows[0, i, pl.ds(g * L, L)] = zero
            return carry

        lax.fori_loop(0, P, zrow, 0)
        p0 = sid * np_ // NS
        p1 = (sid + 1) * np_ // NS

        def zpiece(p, carry):
            pltpu.sync_copy(rows.at[0, pl.ds(0, P)], acc.at[pl.ds(p * P, P)])
            return carry

        lax.fori_loop(p0, p1, zpiece, 0)
        plsc.subcore_barrier()

        def fire_blk(g):
            base = wid * cpt + g * BI
            slot = lax.rem(g, 3)
            pltpu.async_copy(eb_hbm.at[pl.ds(base, BI)], ring.at[slot], sb)
            pltpu.async_copy(wb_hbm.at[pl.ds(base, BI)], wring.at[slot], sb)

        def drain_blk(g):
            slot = lax.rem(g, 3)
            pltpu.make_async_copy(eb_hbm.at[pl.ds(0, BI)],
                                  ring.at[slot], sb).wait()
            pltpu.make_async_copy(wb_hbm.at[pl.ds(0, BI)],
                                  wring.at[slot], sb).wait()

        def fire_gather(r, j, b):
            pltpu.async_copy(x_hbm.at[ring.at[r, j, 0]], rows.at[b], sg[b])

        def drain_gather(b):
            pltpu.make_async_copy(x_hbm.at[pl.ds(0, CE)],
                                  rows.at[b], sg[b]).wait()

        def fire_scatter(r, j, b):
            pltpu.async_copy(rows.at[b], acc.at[ring.at[r, j, 1]], ss[b],
                             add=True)

        def drain_scatter(b):
            pltpu.make_async_copy(x_hbm.at[pl.ds(0, CE)],
                                  rows.at[b], ss[b]).wait()

        def scale(r, j, b):
            def sgroup(t, carry):
                wg = wring[r, j, 0, pl.ds(t * L, L)]
                for jj in range(L):
                    w0 = wg[jj]
                    wv = jnp.full((L,), w0 + w0, jnp.float32)
                    for q in range(d // L):
                        sl = pl.ds(q * L, L)
                        rows[b, t * L + jj, sl] = rows[b, t * L + jj, sl] * wv
                return carry

            lax.fori_loop(0, CE // L, sgroup, 0)

        fire_blk(0)
        drain_blk(0)
        fire_gather(0, 0, 0)
        fire_blk(1)

        def body3(i, carry):
            r = lax.rem(i, 3)
            rn = lax.rem(i + 1, 3)

            @pl.when(i > 0)
            def _():
                drain_scatter(1)

            fire_gather(r, 1, 1)
            drain_gather(0)
            scale(r, 0, 0)
            fire_scatter(r, 0, 0)

            @pl.when(i > 0)
            def _():
                drain_scatter(2)

            fire_gather(r, 2, 2)
            drain_gather(1)
            scale(r, 1, 1)
            fire_scatter(r, 1, 1)

            drain_blk(i + 1)
            drain_scatter(0)

            @pl.when(i + 2 <= nb)
            def _():
                fire_blk(i + 2)

            fire_gather(rn, 0, 0)
            drain_gather(2)
            scale(r, 2, 2)
            fire_scatter(r, 2, 2)
            return carry

        lax.fori_loop(0, nb, body3, 0)
        rl = lax.rem(nb, 3)
        drain_scatter(1)
        fire_gather(rl, 1, 1)
        drain_gather(0)
        scale(rl, 0, 0)
        fire_scatter(rl, 0, 0)
        drain_scatter(2)
        drain_gather(1)
        scale(rl, 1, 1)
        fire_scatter(rl, 1, 1)
        drain_scatter(0)
        drain_scatter(1)
        plsc.subcore_barrier()

        def wpiece(p, carry):
            sl = pl.ds(p * P, P)
            pltpu.sync_copy(acc.at[sl], out_hbm.at[cid, sl])
            return carry

        lax.fori_loop(p0, p1, wpiece, 0)

    return spmm(x, eb, wb)


def kernel(input, edge_index, edge_weight, W):
    srcv = edge_index[1].reshape(-1, CE)
    dstv = edge_index[0].reshape(-1, CE)
    eb = jnp.stack([srcv, dstv], axis=1)
    eb = jnp.concatenate(
        [eb, jnp.zeros((1, 2, CE), jnp.int32)], axis=0)
    wb = edge_weight.reshape(-1, 1, CE)
    wb = jnp.concatenate(
        [wb, jnp.zeros((1, 1, CE), jnp.float32)], axis=0)
    parts = _spmm_sc(input, eb, wb)
    w2 = _mm(W, W, W.shape[0])
    return _mm_final(parts, input, w2, 2000)


# speedup vs baseline: 1.1137x; 1.0523x over previous
import functools

import jax
import jax.numpy as jnp
from jax import lax
from jax.experimental import pallas as pl
from jax.experimental.pallas import tpu as pltpu
from jax.experimental.pallas import tpu_sc as plsc

NC = 2
NS = 16
L = 16
CE = 80
BI = 4
NW = NC * NS


def _mm_body(x_ref, w_ref, o_ref):
    o_ref[...] = jnp.dot(x_ref[...], w_ref[...],
                         preferred_element_type=jnp.float32)


def _mm(x, W, bm):
    n, d = x.shape
    return pl.pallas_call(
        _mm_body,
        grid=(n // bm,),
        in_specs=[pl.BlockSpec((bm, d), lambda i: (i, 0)),
                  pl.BlockSpec((d, d), lambda i: (0, 0))],
        out_specs=pl.BlockSpec((bm, d), lambda i: (i, 0)),
        out_shape=jax.ShapeDtypeStruct((n, d), jnp.float32),
    )(x, W)


def _mm_final_body(p_ref, x_ref, w_ref, o_ref):
    s = p_ref[0] + p_ref[1] - x_ref[...]
    o_ref[...] = jnp.dot(s, w_ref[...], preferred_element_type=jnp.float32)


def _mm_final(parts, x, W2, bm):
    n, d = x.shape
    return pl.pallas_call(
        _mm_final_body,
        grid=(n // bm,),
        in_specs=[pl.BlockSpec((NC, bm, d), lambda i: (0, i, 0)),
                  pl.BlockSpec((bm, d), lambda i: (i, 0)),
                  pl.BlockSpec((d, d), lambda i: (0, 0))],
        out_specs=pl.BlockSpec((bm, d), lambda i: (i, 0)),
        out_shape=jax.ShapeDtypeStruct((n, d), jnp.float32),
    )(parts, x, W2)


def _spmm_sc(x, eb, wb):
    n, d = x.shape
    ept = (eb.shape[0] - BI + 1) * CE // NW
    cpt = ept // CE
    ni = cpt // BI
    P = 80
    np_ = n // P
    mesh = plsc.VectorSubcoreMesh(core_axis_name="c", subcore_axis_name="s",
                                  num_cores=NC, num_subcores=NS)

    @functools.partial(
        pl.kernel,
        out_type=jax.ShapeDtypeStruct((NC, n, d), jnp.float32),
        mesh=mesh,
        scratch_types=[
            pltpu.VMEM((3, BI, 2, CE), jnp.int32),
            pltpu.VMEM((3, BI, 1, CE), jnp.float32),
            pltpu.VMEM((4, CE, d), jnp.float32),
            pltpu.VMEM_SHARED((n, d), jnp.float32),
            pltpu.SemaphoreType.DMA,
            pltpu.SemaphoreType.DMA,
            pltpu.SemaphoreType.DMA,
            pltpu.SemaphoreType.DMA,
            pltpu.SemaphoreType.DMA,
            pltpu.SemaphoreType.DMA,
            pltpu.SemaphoreType.DMA,
            pltpu.SemaphoreType.DMA,
            pltpu.SemaphoreType.DMA,
        ],
    )
    def spmm(x_hbm, eb_hbm, wb_hbm, out_hbm, ring, wring, rows, acc,
             sb, sg0, sg1, sg2, sg3, ss0, ss1, ss2, ss3):
        sg = (sg0, sg1, sg2, sg3)
        ss = (ss0, ss1, ss2, ss3)
        cid = lax.axis_index("c")
        sid = lax.axis_index("s")
        wid = sid * NC + cid

        zero = jnp.zeros((L,), jnp.float32)

        def zrow(i, carry):
            for g in range(d // L):
                rows[0, i, pl.ds(g * L, L)] = zero
            return carry

        lax.fori_loop(0, P, zrow, 0)
        p0 = sid * np_ // NS
        p1 = (sid + 1) * np_ // NS

        def zpiece(p, carry):
            pltpu.sync_copy(rows.at[0, pl.ds(0, P)], acc.at[pl.ds(p * P, P)])
            return carry

        lax.fori_loop(p0, p1, zpiece, 0)
        plsc.subcore_barrier()

        def fire_blk(g):
            base = wid * cpt + g * BI
            slot = lax.rem(g, 3)
            pltpu.async_copy(eb_hbm.at[pl.ds(base, BI)], ring.at[slot], sb)
            pltpu.async_copy(wb_hbm.at[pl.ds(base, BI)], wring.at[slot], sb)

        def drain_blk(g):
            slot = lax.rem(g, 3)
            pltpu.make_async_copy(eb_hbm.at[pl.ds(0, BI)],
                                  ring.at[slot], sb).wait()
            pltpu.make_async_copy(wb_hbm.at[pl.ds(0, BI)],
                                  wring.at[slot], sb).wait()

        def fire_gather(r, j, b):
            pltpu.async_copy(x_hbm.at[ring.at[r, j, 0]], rows.at[b], sg[b])

        def drain_gather(b):
            pltpu.make_async_copy(x_hbm.at[pl.ds(0, CE)],
                                  rows.at[b], sg[b]).wait()

        def fire_scatter(r, j, b):
            pltpu.async_copy(rows.at[b], acc.at[ring.at[r, j, 1]], ss[b],
                             add=True)

        def drain_scatter(b):
            pltpu.make_async_copy(x_hbm.at[pl.ds(0, CE)],
                                  rows.at[b], ss[b]).wait()

        def scale(r, j, b):
            def sgroup(t, carry):
                wg = wring[r, j, 0, pl.ds(t * L, L)]
                for jj in range(L):
                    w0 = wg[jj]
                    wv = jnp.full((L,), w0 + w0, jnp.float32)
                    for q in range(d // L):
                        sl = pl.ds(q * L, L)
                        rows[b, t * L + jj, sl] = rows[b, t * L + jj, sl] * wv
                return carry

            lax.fori_loop(0, CE // L, sgroup, 0)

        fire_blk(0)
        drain_blk(0)
        fire_gather(0, 0, 0)
        fire_gather(0, 1, 1)
        fire_blk(1)

        def body4(i, carry):
            r = lax.rem(i, 3)
            rn = lax.rem(i + 1, 3)

            @pl.when(i > 0)
            def _():
                drain_scatter(2)

            fire_gather(r, 2, 2)
            drain_gather(0)
            scale(r, 0, 0)
            fire_scatter(r, 0, 0)

            @pl.when(i > 0)
            def _():
                drain_scatter(3)

            fire_gather(r, 3, 3)
            drain_gather(1)
            scale(r, 1, 1)
            fire_scatter(r, 1, 1)

            drain_blk(i + 1)

            @pl.when(i + 2 <= ni)
            def _():
                fire_blk(i + 2)

            drain_scatter(0)
            fire_gather(rn, 0, 0)
            drain_gather(2)
            scale(r, 2, 2)
            fire_scatter(r, 2, 2)

            drain_scatter(1)

            @pl.when(i + 1 < ni)
            def _():
                fire_gather(rn, 1, 1)

            drain_gather(3)
            scale(r, 3, 3)
            fire_scatter(r, 3, 3)
            return carry

        lax.fori_loop(0, ni, body4, 0)
        rl = lax.rem(ni, 3)
        drain_scatter(2)
        drain_gather(0)
        scale(rl, 0, 0)
        fire_scatter(rl, 0, 0)
        drain_scatter(3)
        drain_scatter(0)
        plsc.subcore_barrier()

        def wpiece(p, carry):
            sl = pl.ds(p * P, P)
            pltpu.sync_copy(acc.at[sl], out_hbm.at[cid, sl])
            return carry

        lax.fori_loop(p0, p1, wpiece, 0)

    return spmm(x, eb, wb)


def kernel(input, edge_index, edge_weight, W):
    srcv = edge_index[1].reshape(-1, CE)
    dstv = edge_index[0].reshape(-1, CE)
    eb = jnp.stack([srcv, dstv], axis=1)
    eb = jnp.concatenate(
        [eb, jnp.zeros((BI - 1, 2, CE), jnp.int32)], axis=0)
    wb = edge_weight.reshape(-1, 1, CE)
    wb = jnp.concatenate(
        [wb, jnp.zeros((BI - 1, 1, CE), jnp.float32)], axis=0)
    parts = _spmm_sc(input, eb, wb)
    w2 = _mm(W, W, W.shape[0])
    return _mm_final(parts, input, w2, 2000)
